# Initial kernel scaffold; baseline (speedup 1.0000x reference)
import jax
import jax.numpy as jnp
from jax.experimental import pallas as pl


def kernel(x1, x2, emb, W1, b1, W2, b2):
    raise NotImplementedError("write your pallas kernel here")


# speedup vs baseline: 16.0329x; 16.0329x over previous
import functools

import jax
import jax.numpy as jnp
from jax import lax
from jax.experimental import pallas as pl
from jax.experimental.pallas import tpu as pltpu
from jax.experimental.pallas import tpu_sc as plsc

BATCH = 16384
LIN_IN = 13
N_CATS = 26
EMB_DIM = 16
HIDDEN = 256
OUT = 6

TOTAL = BATCH * N_CATS
NC, NS = 2, 16
NW = NC * NS
PER_W = TOTAL // NW
GROUP = 128
GROUPS_PER_W = PER_W // GROUP
GROUPS_PER_CHUNK = 13
CHUNK = GROUPS_PER_CHUNK * GROUP
CHUNKS_PER_W = GROUPS_PER_W // GROUPS_PER_CHUNK


def _gather_body(emb_hbm, idx_hbm, out_hbm, idx_v, rows_v, sem):
    wid = lax.axis_index("s") * NC + lax.axis_index("c")
    row0 = wid * GROUPS_PER_W
    base = wid * PER_W
    pltpu.sync_copy(idx_hbm.at[pl.ds(row0, GROUPS_PER_W)], idx_v)

    def chunk_body(s, carry):
        copies = []
        for j in range(GROUPS_PER_CHUNK):
            c = pltpu.async_copy(
                emb_hbm.at[idx_v.at[s * GROUPS_PER_CHUNK + j]],
                rows_v.at[pl.ds(j * GROUP, GROUP)],
                sem,
            )
            copies.append(c)
        for c in copies:
            c.wait()
        pltpu.sync_copy(rows_v, out_hbm.at[pl.ds(base + s * CHUNK, CHUNK)])
        return carry

    lax.fori_loop(0, CHUNKS_PER_W, chunk_body, 0)


_gather = functools.partial(
    pl.kernel,
    mesh=plsc.VectorSubcoreMesh(core_axis_name="c", subcore_axis_name="s"),
    compiler_params=pltpu.CompilerParams(use_tc_tiling_on_sc=False),
    out_type=jax.ShapeDtypeStruct((TOTAL, EMB_DIM), jnp.float32),
    scratch_types=[
        pltpu.VMEM((GROUPS_PER_W, GROUP), jnp.int32),
        pltpu.VMEM((CHUNK, EMB_DIM), jnp.float32),
        pltpu.SemaphoreType.DMA,
    ],
)(_gather_body)


BB = 512


def _mlp_body(x1_ref, e_ref, w1a_ref, w1b_ref, b1_ref, w2_ref, b2_ref, out_ref):
    h = jnp.tanh(
        jnp.dot(x1_ref[...], w1a_ref[...], preferred_element_type=jnp.float32)
        + jnp.dot(e_ref[...], w1b_ref[...], preferred_element_type=jnp.float32)
        + b1_ref[...]
    )
    out_ref[...] = (
        jnp.dot(h, w2_ref[...], preferred_element_type=jnp.float32) + b2_ref[...]
    )


def _mlp(x1, e, w1a, w1b, b1, w2, b2):
    grid = (BATCH // BB,)
    return pl.pallas_call(
        _mlp_body,
        grid=grid,
        in_specs=[
            pl.BlockSpec((BB, LIN_IN), lambda i: (i, 0)),
            pl.BlockSpec((BB, N_CATS * EMB_DIM), lambda i: (i, 0)),
            pl.BlockSpec((LIN_IN, HIDDEN), lambda i: (0, 0)),
            pl.BlockSpec((N_CATS * EMB_DIM, HIDDEN), lambda i: (0, 0)),
            pl.BlockSpec((1, HIDDEN), lambda i: (0, 0)),
            pl.BlockSpec((HIDDEN, OUT), lambda i: (0, 0)),
            pl.BlockSpec((1, OUT), lambda i: (0, 0)),
        ],
        out_specs=pl.BlockSpec((BB, OUT), lambda i: (i, 0)),
        out_shape=jax.ShapeDtypeStruct((BATCH, OUT), jnp.float32),
    )(x1, e, w1a, w1b, b1, w2, b2)


def kernel(x1, x2, emb, W1, b1, W2, b2):
    idx = x2.astype(jnp.int32).reshape(TOTAL // GROUP, GROUP)
    e = _gather(emb, idx).reshape(BATCH, N_CATS * EMB_DIM)
    return _mlp(
        x1,
        e,
        W1[:LIN_IN],
        W1[LIN_IN:],
        b1.reshape(1, HIDDEN),
        W2,
        b2.reshape(1, OUT),
    )
